# Initial kernel scaffold; baseline (speedup 1.0000x reference)
import jax
import jax.numpy as jnp
from jax import lax
from jax.experimental import pallas as pl
from jax.experimental.pallas import tpu as pltpu
from jax.experimental.pallas import tpu_sc as plsc

BATCH = 4096
SEQ = 200
EMBED = 32
NC = 2
NS = 16
NW = NC * NS
BPW = BATCH // NW
IPW = BPW * SEQ
CHUNK = 128
NCHUNK = IPW // CHUNK


def _pool_body(x_hbm, dst_hbm, emb_hbm, out_hbm,
               idx_v, dstidx_v, buf0, buf1, slab_sh, sem0, sem1):
    c = lax.axis_index("c")
    s = lax.axis_index("s")
    wid = s * NC + c

    pltpu.sync_copy(x_hbm.at[wid], idx_v)
    pltpu.sync_copy(dst_hbm.at[s], dstidx_v)

    def _zero(i, _):
        z = jnp.zeros((16,), jnp.float32)
        buf0[i // 2, pl.ds((i % 2) * 16, 16)] = z
        return 0
    lax.fori_loop(0, BPW * 2, _zero, 0)
    pltpu.sync_copy(buf0, slab_sh.at[pl.ds(s * BPW, BPW)])

    def _start(chunk, buf, sem):
        return pltpu.async_copy(emb_hbm.at[idx_v.at[chunk]], buf, sem)

    def _wait(chunk, buf, sem):
        pltpu.make_async_copy(emb_hbm.at[idx_v.at[chunk]], buf, sem).wait()

    def _scadd(chunk, buf):
        pltpu.sync_copy(buf, slab_sh.at[dstidx_v.at[chunk]], add=True)

    def body(i, _):
        a = 2 * i
        b = 2 * i + 1
        _start(b, buf1, sem1)
        _wait(a, buf0, sem0)
        _scadd(a, buf0)

        @pl.when(i < NCHUNK // 2 - 1)
        def _():
            _start(a + 2, buf0, sem0)

        _wait(b, buf1, sem1)
        _scadd(b, buf1)
        return 0

    _start(0, buf0, sem0)
    lax.fori_loop(0, NCHUNK // 2, body, 0)

    pltpu.sync_copy(slab_sh.at[pl.ds(s * BPW, BPW)],
                    out_hbm.at[pl.ds(wid * BPW, BPW)])


def _sc_pool(x_r, dst, embedding):
    mesh = plsc.VectorSubcoreMesh(core_axis_name="c", subcore_axis_name="s")
    return pl.kernel(
        _pool_body,
        out_type=jax.ShapeDtypeStruct((BATCH, EMBED), jnp.float32),
        mesh=mesh,
        scratch_types=[
            pltpu.VMEM((NCHUNK, CHUNK), jnp.int32),
            pltpu.VMEM((NCHUNK, CHUNK), jnp.int32),
            pltpu.VMEM((CHUNK, EMBED), jnp.float32),
            pltpu.VMEM((CHUNK, EMBED), jnp.float32),
            pltpu.VMEM_SHARED((NS * BPW, EMBED), jnp.float32),
            pltpu.SemaphoreType.DMA,
            pltpu.SemaphoreType.DMA,
        ],
    )(x_r, dst, embedding)


def _mlp_body(p_ref, w1_ref, b1_ref, w2_ref, b2_ref, o_ref):
    p = p_ref[...] * jnp.float32(1.0 / SEQ)
    h = jnp.dot(p, w1_ref[...], preferred_element_type=jnp.float32)
    h = jnp.maximum(h + b1_ref[...], 0.0)
    o_ref[...] = jnp.dot(h, w2_ref[...],
                         preferred_element_type=jnp.float32) + b2_ref[...]


def _tc_mlp(pooled, fc1_w, fc1_b, fc2_w, fc2_b):
    return pl.pallas_call(
        _mlp_body,
        out_shape=jax.ShapeDtypeStruct((BATCH, fc2_w.shape[1]), jnp.float32),
    )(pooled, fc1_w, fc1_b.reshape(1, -1), fc2_w, fc2_b.reshape(1, -1))


def kernel(x, embedding, fc1_w, fc1_b, fc2_w, fc2_b):
    x_r = x.reshape(NW, NCHUNK, CHUNK)
    p = jnp.arange(IPW, dtype=jnp.int32) // SEQ
    dst = (jnp.arange(NS, dtype=jnp.int32)[:, None] * BPW + p[None, :])
    dst = dst.reshape(NS, NCHUNK, CHUNK)
    sums = _sc_pool(x_r, dst, embedding)
    return _tc_mlp(sums, fc1_w, fc1_b, fc2_w, fc2_b)


# speedup vs baseline: 2.1274x; 2.1274x over previous
import jax
import jax.numpy as jnp
from jax import lax
from jax.experimental import pallas as pl
from jax.experimental.pallas import tpu as pltpu
from jax.experimental.pallas import tpu_sc as plsc

BATCH = 4096
SEQ = 200
EMBED = 32
NC = 2
NS = 16
NW = NC * NS
BPW = BATCH // NW
IPW = BPW * SEQ
CHUNK = 128
NCHUNK = IPW // CHUNK


def _pool_body(x_hbm, dst_hbm, emb_hbm, out_hbm,
               idx_v, dstidx_v, buf0, buf1, slab_sh, sem0, sem1):
    c = lax.axis_index("c")
    s = lax.axis_index("s")
    wid = s * NC + c

    pltpu.sync_copy(x_hbm.at[wid], idx_v)
    pltpu.sync_copy(dst_hbm.at[s], dstidx_v)

    def _zero(i, _):
        z = jnp.zeros((16,), jnp.float32)
        buf0[i // 2, pl.ds((i % 2) * 16, 16)] = z
        return 0
    lax.fori_loop(0, BPW * 2, _zero, 0)
    pltpu.sync_copy(buf0, slab_sh.at[pl.ds(s * BPW, BPW)])

    def _start(chunk, buf, sem):
        return pltpu.async_copy(emb_hbm.at[idx_v.at[chunk]], buf, sem)

    def _wait(chunk, buf, sem):
        pltpu.make_async_copy(emb_hbm.at[idx_v.at[chunk]], buf, sem).wait()

    def _scadd(chunk, buf):
        pltpu.sync_copy(buf, slab_sh.at[dstidx_v.at[chunk]], add=True)

    def body(i, _):
        a = 2 * i
        b = 2 * i + 1
        _start(b, buf1, sem1)
        _wait(a, buf0, sem0)
        _scadd(a, buf0)

        @pl.when(i < NCHUNK // 2 - 1)
        def _():
            _start(a + 2, buf0, sem0)

        _wait(b, buf1, sem1)
        _scadd(b, buf1)
        return 0

    _start(0, buf0, sem0)
    lax.fori_loop(0, NCHUNK // 2, body, 0)

    pltpu.sync_copy(slab_sh.at[pl.ds(s * BPW, BPW)],
                    out_hbm.at[pl.ds(wid * BPW, BPW)])


def _sc_pool(x_r, dst, embedding):
    mesh = plsc.VectorSubcoreMesh(core_axis_name="c", subcore_axis_name="s")
    return pl.kernel(
        _pool_body,
        out_type=jax.ShapeDtypeStruct((BATCH, EMBED), jnp.float32),
        mesh=mesh,
        compiler_params=pltpu.CompilerParams(use_tc_tiling_on_sc=False),
        scratch_types=[
            pltpu.VMEM((NCHUNK, CHUNK), jnp.int32),
            pltpu.VMEM((NCHUNK, CHUNK), jnp.int32),
            pltpu.VMEM((CHUNK, EMBED), jnp.float32),
            pltpu.VMEM((CHUNK, EMBED), jnp.float32),
            pltpu.VMEM_SHARED((NS * BPW, EMBED), jnp.float32),
            pltpu.SemaphoreType.DMA,
            pltpu.SemaphoreType.DMA,
        ],
    )(x_r, dst, embedding)


def _mlp_body(p_ref, w1_ref, b1_ref, w2_ref, b2_ref, o_ref):
    p = p_ref[...] * jnp.float32(1.0 / SEQ)
    h = jnp.dot(p, w1_ref[...], preferred_element_type=jnp.float32,
                precision=lax.Precision.HIGHEST)
    h = jnp.maximum(h + b1_ref[...], 0.0)
    o_ref[...] = jnp.dot(h, w2_ref[...], preferred_element_type=jnp.float32,
                         precision=lax.Precision.HIGHEST) + b2_ref[...]


def _tc_mlp(pooled, fc1_w, fc1_b, fc2_w, fc2_b):
    return pl.pallas_call(
        _mlp_body,
        out_shape=jax.ShapeDtypeStruct((BATCH, fc2_w.shape[1]), jnp.float32),
    )(pooled, fc1_w, fc1_b.reshape(1, -1), fc2_w, fc2_b.reshape(1, -1))


def kernel(x, embedding, fc1_w, fc1_b, fc2_w, fc2_b):
    x_r = x.reshape(NW, NCHUNK, CHUNK)
    p = jnp.arange(IPW, dtype=jnp.int32) // SEQ
    dst = (jnp.arange(NS, dtype=jnp.int32)[:, None] * BPW + p[None, :])
    dst = dst.reshape(NS, NCHUNK, CHUNK)
    sums = _sc_pool(x_r, dst, embedding)
    return _tc_mlp(sums, fc1_w, fc1_b, fc2_w, fc2_b)


# speedup vs baseline: 2.1679x; 1.0190x over previous
import jax
import jax.numpy as jnp
from jax import lax
from jax.experimental import pallas as pl
from jax.experimental.pallas import tpu as pltpu
from jax.experimental.pallas import tpu_sc as plsc

BATCH = 4096
SEQ = 200
EMBED = 32
NC = 2
NS = 16
NW = NC * NS
BPW = BATCH // NW


def _pool_body(xt_hbm, dst_hbm, emb_hbm, out_hbm,
               idx_v, dst_v, buf0, buf1, slab_sh, sem0, sem1):
    c = lax.axis_index("c")
    s = lax.axis_index("s")
    wid = s * NC + c

    pltpu.sync_copy(xt_hbm.at[:, pl.ds(wid * BPW, BPW)], idx_v)
    pltpu.sync_copy(dst_hbm.at[s], dst_v)

    def _zero(i, _):
        z = jnp.zeros((16,), jnp.float32)
        buf0[i, pl.ds(0, 16)] = z
        buf0[i, pl.ds(16, 16)] = z
        return 0
    lax.fori_loop(0, BPW, _zero, 0)
    pltpu.sync_copy(buf0, slab_sh.at[pl.ds(s * BPW, BPW)])

    def _start(chunk, buf, sem):
        return pltpu.async_copy(emb_hbm.at[idx_v.at[chunk]], buf, sem)

    def _wait(chunk, buf, sem):
        pltpu.make_async_copy(emb_hbm.at[idx_v.at[chunk]], buf, sem).wait()

    def _scadd(buf):
        pltpu.sync_copy(buf, slab_sh.at[dst_v], add=True)

    def body(i, _):
        a = 2 * i
        b = 2 * i + 1
        _start(b, buf1, sem1)
        _wait(a, buf0, sem0)
        _scadd(buf0)

        @pl.when(i < SEQ // 2 - 1)
        def _():
            _start(a + 2, buf0, sem0)

        _wait(b, buf1, sem1)
        _scadd(buf1)
        return 0

    _start(0, buf0, sem0)
    lax.fori_loop(0, SEQ // 2, body, 0)

    pltpu.sync_copy(slab_sh.at[pl.ds(s * BPW, BPW)],
                    out_hbm.at[pl.ds(wid * BPW, BPW)])


def _sc_pool(xt, dst, embedding):
    mesh = plsc.VectorSubcoreMesh(core_axis_name="c", subcore_axis_name="s")
    return pl.kernel(
        _pool_body,
        out_type=jax.ShapeDtypeStruct((BATCH, EMBED), jnp.float32),
        mesh=mesh,
        compiler_params=pltpu.CompilerParams(use_tc_tiling_on_sc=False),
        scratch_types=[
            pltpu.VMEM((SEQ, BPW), jnp.int32),
            pltpu.VMEM((BPW,), jnp.int32),
            pltpu.VMEM((BPW, EMBED), jnp.float32),
            pltpu.VMEM((BPW, EMBED), jnp.float32),
            pltpu.VMEM_SHARED((NS * BPW, EMBED), jnp.float32),
            pltpu.SemaphoreType.DMA,
            pltpu.SemaphoreType.DMA,
        ],
    )(xt, dst, embedding)


def _mlp_body(p_ref, w1_ref, b1_ref, w2_ref, b2_ref, o_ref):
    p = p_ref[...] * jnp.float32(1.0 / SEQ)
    h = jnp.dot(p, w1_ref[...], preferred_element_type=jnp.float32,
                precision=lax.Precision.HIGHEST)
    h = jnp.maximum(h + b1_ref[...], 0.0)
    o_ref[...] = jnp.dot(h, w2_ref[...], preferred_element_type=jnp.float32,
                         precision=lax.Precision.HIGHEST) + b2_ref[...]


def _tc_mlp(pooled, fc1_w, fc1_b, fc2_w, fc2_b):
    return pl.pallas_call(
        _mlp_body,
        out_shape=jax.ShapeDtypeStruct((BATCH, fc2_w.shape[1]), jnp.float32),
    )(pooled, fc1_w, fc1_b.reshape(1, -1), fc2_w, fc2_b.reshape(1, -1))


def kernel(x, embedding, fc1_w, fc1_b, fc2_w, fc2_b):
    xt = x.T
    dst = (jnp.arange(NS, dtype=jnp.int32)[:, None] * BPW
           + jnp.arange(BPW, dtype=jnp.int32)[None, :])
    sums = _sc_pool(xt, dst, embedding)
    return _tc_mlp(sums, fc1_w, fc1_b, fc2_w, fc2_b)


# speedup vs baseline: 3.2131x; 1.4822x over previous
import jax
import jax.numpy as jnp
from jax import lax
from jax.experimental import pallas as pl
from jax.experimental.pallas import tpu as pltpu
from jax.experimental.pallas import tpu_sc as plsc
from jax.experimental import layout as jlayout

BATCH = 4096
SEQ = 200
EMBED = 32
NC = 2
NS = 16
NW = NC * NS
BPW = BATCH // NW


def _pool_body(xt_hbm, dst_hbm, emb_hbm, out_hbm,
               idx_v, dst_v, buf0, buf1, slab_sh, sem0, sem1):
    c = lax.axis_index("c")
    s = lax.axis_index("s")
    wid = s * NC + c

    pltpu.sync_copy(xt_hbm.at[:, pl.ds(wid * BPW, BPW)], idx_v)
    pltpu.sync_copy(dst_hbm.at[s], dst_v)

    def _zero(i, _):
        z = jnp.zeros((16,), jnp.float32)
        buf0[i, pl.ds(0, 16)] = z
        buf0[i, pl.ds(16, 16)] = z
        return 0
    lax.fori_loop(0, BPW, _zero, 0)
    pltpu.sync_copy(buf0, slab_sh.at[pl.ds(s * BPW, BPW)])

    def _start(chunk, buf, sem):
        return pltpu.async_copy(emb_hbm.at[idx_v.at[chunk]], buf, sem)

    def _wait(chunk, buf, sem):
        pltpu.make_async_copy(emb_hbm.at[idx_v.at[chunk]], buf, sem).wait()

    def _scadd(buf):
        pltpu.sync_copy(buf, slab_sh.at[dst_v], add=True)

    def body(i, _):
        a = 2 * i
        b = 2 * i + 1
        _start(b, buf1, sem1)
        _wait(a, buf0, sem0)
        _scadd(buf0)

        @pl.when(i < SEQ // 2 - 1)
        def _():
            _start(a + 2, buf0, sem0)

        _wait(b, buf1, sem1)
        _scadd(buf1)
        return 0

    _start(0, buf0, sem0)
    lax.fori_loop(0, SEQ // 2, body, 0)

    pltpu.sync_copy(slab_sh.at[pl.ds(s * BPW, BPW)],
                    out_hbm.at[pl.ds(wid * BPW, BPW)])


def _sc_pool(xt, dst, embedding):
    mesh = plsc.VectorSubcoreMesh(core_axis_name="c", subcore_axis_name="s")
    return pl.kernel(
        _pool_body,
        out_type=jax.ShapeDtypeStruct((BATCH, EMBED), jnp.float32),
        mesh=mesh,
        compiler_params=pltpu.CompilerParams(use_tc_tiling_on_sc=False),
        scratch_types=[
            pltpu.VMEM((SEQ, BPW), jnp.int32),
            pltpu.VMEM((BPW,), jnp.int32),
            pltpu.VMEM((BPW, EMBED), jnp.float32),
            pltpu.VMEM((BPW, EMBED), jnp.float32),
            pltpu.VMEM_SHARED((NS * BPW, EMBED), jnp.float32),
            pltpu.SemaphoreType.DMA,
            pltpu.SemaphoreType.DMA,
        ],
    )(xt, dst, embedding)


def _mlp_body(p_ref, w1_ref, b1_ref, w2_ref, b2_ref, o_ref):
    p = p_ref[...] * jnp.float32(1.0 / SEQ)
    h = jnp.dot(p, w1_ref[...], preferred_element_type=jnp.float32,
                precision=lax.Precision.HIGHEST)
    h = jnp.maximum(h + b1_ref[...], 0.0)
    o_ref[...] = jnp.dot(h, w2_ref[...], preferred_element_type=jnp.float32,
                         precision=lax.Precision.HIGHEST) + b2_ref[...]


def _tc_mlp(pooled, fc1_w, fc1_b, fc2_w, fc2_b):
    return pl.pallas_call(
        _mlp_body,
        out_shape=jax.ShapeDtypeStruct((BATCH, fc2_w.shape[1]), jnp.float32),
    )(pooled, fc1_w, fc1_b.reshape(1, -1), fc2_w, fc2_b.reshape(1, -1))


def kernel(x, embedding, fc1_w, fc1_b, fc2_w, fc2_b):
    xt = x.T
    dst = (jnp.arange(NS, dtype=jnp.int32)[:, None] * BPW
           + jnp.arange(BPW, dtype=jnp.int32)[None, :])
    emb_lin = jlayout.with_layout_constraint(
        embedding,
        jlayout.Layout(major_to_minor=(0, 1), tiling=((8,),)))
    sums = _sc_pool(xt, dst, emb_lin)
    return _tc_mlp(sums, fc1_w, fc1_b, fc2_w, fc2_b)
